# baseline (speedup 1.0000x reference)
import functools

import jax
import jax.numpy as jnp
from jax import lax
from jax.experimental import pallas as pl
from jax.experimental.pallas import tpu as pltpu
from jax.experimental.pallas import tpu_sc as plsc

VOCAB = 1000000
EMB = 64
B = 4096
L = 200
N = B * L

NC = 2
NS = 16
NW = NC * NS
BLK = 512
NBLK = B // BLK
MAXL = (L + NW - 1) // NW

ET = EMB // 8
BT = B // 128
BTB = BLK // 128


def _make_fused_kernel():
    mesh = plsc.VectorSubcoreMesh(core_axis_name="c", subcore_axis_name="s")

    @functools.partial(
        pl.kernel,
        mesh=mesh,
        out_type=jax.ShapeDtypeStruct((L, ET, BT, 8, 128), jnp.float32),
        compiler_params=pltpu.CompilerParams(
            use_tc_tiling_on_sc=False, needs_layout_passes=False
        ),
        scratch_types=[
            pltpu.VMEM((B,), jnp.int32),
            pltpu.VMEM((BLK,), jnp.int32),
            pltpu.VMEM((BLK,), jnp.int32),
            pltpu.VMEM((BLK, 2 * EMB), jnp.float32),
            pltpu.VMEM((ET, BTB, 8, 128), jnp.float32),
            pltpu.SemaphoreType.DMA,
        ],
    )
    def fused_kernel(tab_hbm, idxt_hbm, out_hbm, idx_l, rowix, halfoff, rows_v,
                     outt, sem):
        wid = lax.axis_index("s") * NC + lax.axis_index("c")
        iota16 = lax.iota(jnp.int32, 16)

        def pos_body(t, carry):
            l = wid + t * NW

            @pl.when(l < L)
            def _():
                pltpu.sync_copy(idxt_hbm.at[l], idx_l)
                for blk in range(NBLK):
                    b0 = blk * BLK

                    def prep(j0, c):
                        v = idx_l[pl.ds(b0 + j0 * 16, 16)]
                        rowix[pl.ds(j0 * 16, 16)] = lax.shift_right_logical(v, 1)
                        halfoff[pl.ds(j0 * 16, 16)] = (v & 1) * EMB
                        return c

                    lax.fori_loop(0, BLK // 16, prep, 0)
                    pltpu.async_copy(tab_hbm.at[rowix], rows_v, sem).wait()

                    def transpose(j0, c):
                        j16 = j0 * 16 + iota16
                        bt16 = lax.shift_right_logical(j16, 7)
                        bs16 = j16 & 127
                        colb = halfoff[pl.ds(j0 * 16, 16)]
                        for e in range(EMB):
                            vec = plsc.load_gather(rows_v, [j16, colb + e])
                            et16 = jnp.full((16,), e // 8, jnp.int32)
                            es16 = jnp.full((16,), e % 8, jnp.int32)
                            plsc.store_scatter(outt, [et16, bt16, es16, bs16], vec)
                        return c

                    lax.fori_loop(0, BLK // 16, transpose, 0)
                    pltpu.sync_copy(outt, out_hbm.at[l, :, pl.ds(blk * BTB, BTB)])

            return carry

        lax.fori_loop(0, MAXL, pos_body, 0)

    return fused_kernel


_fused = _make_fused_kernel()


def kernel(mode, word_inputs, word_seq_lengths, table):
    tab_pairs = table.reshape(VOCAB // 2, 2 * EMB)
    idx_t = word_inputs.astype(jnp.int32).T
    out5 = _fused(tab_pairs, idx_t)
    return out5.transpose(2, 4, 0, 1, 3).reshape(B, L, EMB)


# speedup vs baseline: 1.6229x; 1.6229x over previous
import functools

import jax
import jax.numpy as jnp
from jax import lax
from jax.experimental import pallas as pl
from jax.experimental.pallas import tpu as pltpu
from jax.experimental.pallas import tpu_sc as plsc

VOCAB = 1000000
EMB = 64
B = 4096
L = 200
N = B * L

NC = 2
NS = 16
NW = NC * NS
BLK = 512
TASKS = N // BLK
PER_W = TASKS // NW
ET = EMB // 8
SLAB = BLK * EMB
LSLAB = B * EMB

_ADDR = [[(e // 8) * (BLK * 8) + (e % 8) * 128 for e in range(e0, e0 + 16)]
         for e0 in range(0, EMB, 16)]


def _make_fused_kernel():
    mesh = plsc.VectorSubcoreMesh(core_axis_name="c", subcore_axis_name="s")

    @functools.partial(
        pl.kernel,
        mesh=mesh,
        out_type=jax.ShapeDtypeStruct((L * EMB * B,), jnp.float32),
        compiler_params=pltpu.CompilerParams(
            use_tc_tiling_on_sc=False, needs_layout_passes=False
        ),
        scratch_types=[
            pltpu.VMEM((BLK,), jnp.int32),
            pltpu.VMEM((BLK,), jnp.int32),
            pltpu.VMEM((BLK, EMB), jnp.float32),
            pltpu.VMEM((BLK, EMB), jnp.float32),
            pltpu.VMEM((SLAB,), jnp.float32),
            pltpu.SemaphoreType.DMA,
            pltpu.SemaphoreType.DMA,
        ],
    )
    def fused_kernel(tab_hbm, idx_hbm, out_hbm, idx_a, idx_b, rows_a, rows_b,
                     outt, sem_a, sem_b):
        wid = lax.axis_index("s") * NC + lax.axis_index("c")
        k0 = wid * PER_W
        io16 = lax.iota(jnp.int32, 16)
        addr_vecs = []
        for e0 in range(0, EMB, 16):
            e16 = e0 + io16
            addr_vecs.append(
                lax.shift_right_logical(e16, 3) * (BLK * 8) + (e16 & 7) * 128
            )

        def stage(k, idx_v, rows_v, sem):
            pltpu.sync_copy(idx_hbm.at[pl.ds(k * BLK, BLK)], idx_v)
            return pltpu.async_copy(tab_hbm.at[idx_v], rows_v, sem)

        def consume(k, rows_v):
            @plsc.parallel_loop(0, BLK, unroll=4)
            def _(j):
                jpart = (
                    lax.shift_right_logical(j, 7) * (8 * 128) + (j & 127)
                ).astype(jnp.int32)
                for q in range(EMB // 16):
                    vec = rows_v[j, pl.ds(q * 16, 16)]
                    plsc.store_scatter(outt, [addr_vecs[q] + jpart], vec)

            l = lax.shift_right_logical(k, 3)
            blk = k & 7
            for et in range(ET):
                dst = l * LSLAB + et * (B * 8) + blk * (BLK * 8)
                pltpu.sync_copy(
                    outt.at[pl.ds(et * BLK * 8, BLK * 8)],
                    out_hbm.at[pl.ds(dst, BLK * 8)],
                )

        def pair_body(i, carry):
            k = k0 + i * 2
            cp_b = stage(k + 1, idx_b, rows_b, sem_b)
            pltpu.make_async_copy(tab_hbm.at[pl.ds(0, BLK)], rows_a, sem_a).wait()
            consume(k, rows_a)

            @pl.when(i < PER_W // 2 - 1)
            def _():
                stage(k + 2, idx_a, rows_a, sem_a)

            cp_b.wait()
            consume(k + 1, rows_b)
            return carry

        stage(k0, idx_a, rows_a, sem_a)
        lax.fori_loop(0, PER_W // 2, pair_body, 0)

    return fused_kernel


_fused = _make_fused_kernel()


def kernel(mode, word_inputs, word_seq_lengths, table):
    idx_f = word_inputs.astype(jnp.int32).T.reshape(N)
    out_flat = _fused(table, idx_f)
    return out_flat.reshape(L, ET, B // 128, 8, 128).transpose(
        2, 4, 0, 1, 3
    ).reshape(B, L, EMB)


# speedup vs baseline: 2.6763x; 1.6491x over previous
import functools

import jax
import jax.numpy as jnp
from jax import lax
from jax.experimental import pallas as pl
from jax.experimental.pallas import tpu as pltpu
from jax.experimental.pallas import tpu_sc as plsc

VOCAB = 1000000
EMB = 64
B = 4096
L = 200
N = B * L

NC = 2
NS = 16
NW = NC * NS
BLK = 512
TASKS = N // BLK
PER_W = TASKS // NW
ET = EMB // 8
BT = BLK // 128
SROWS = ET * BT * 8
SPAD = 130
OROWS = L * EMB * B // 128


def _make_fused_kernel():
    mesh = plsc.VectorSubcoreMesh(core_axis_name="c", subcore_axis_name="s")

    @functools.partial(
        pl.kernel,
        mesh=mesh,
        out_type=jax.ShapeDtypeStruct((OROWS, 128), jnp.float32),
        compiler_params=pltpu.CompilerParams(
            use_tc_tiling_on_sc=False, needs_layout_passes=False
        ),
        scratch_types=[
            pltpu.VMEM((BLK,), jnp.int32),
            pltpu.VMEM((BLK,), jnp.int32),
            pltpu.VMEM((BLK, EMB), jnp.float32),
            pltpu.VMEM((BLK, EMB), jnp.float32),
            pltpu.VMEM((SROWS, SPAD), jnp.float32),
            pltpu.SemaphoreType.DMA,
            pltpu.SemaphoreType.DMA,
        ],
    )
    def fused_kernel(tab_hbm, idx_hbm, out_hbm, idx_a, idx_b, rows_a, rows_b,
                     stg, sem_a, sem_b):
        wid = lax.axis_index("s") * NC + lax.axis_index("c")
        k0 = wid * PER_W
        io16 = lax.iota(jnp.int32, 16)
        rowc = []
        for e0 in range(0, EMB, 16):
            e16 = e0 + io16
            rowc.append(lax.shift_right_logical(e16, 3) * (BT * 8) + (e16 & 7))

        def stage(k, idx_v, rows_v, sem):
            pltpu.sync_copy(idx_hbm.at[pl.ds(k * BLK, BLK)], idx_v)
            return pltpu.async_copy(tab_hbm.at[idx_v], rows_v, sem)

        def consume(k, rows_v):
            @plsc.parallel_loop(0, BLK, unroll=4)
            def _(j):
                bt8 = lax.shift_right_logical(j, 7) * 8
                col = jnp.full((16,), 0, jnp.int32) + (j & 127)
                for q in range(EMB // 16):
                    vec = rows_v[j, pl.ds(q * 16, 16)]
                    plsc.store_scatter(stg, [rowc[q] + bt8, col], vec)

            l = lax.shift_right_logical(k, 3)
            blk = k & 7
            for et in range(ET):
                for bt in range(BT):
                    srow = (et * BT + bt) * 8
                    drow = l * (EMB * B // 128) + et * (8 * B // 128) \
                        + (blk * BT + bt) * 8
                    pltpu.sync_copy(
                        stg.at[pl.ds(srow, 8), pl.ds(0, 128)],
                        out_hbm.at[pl.ds(drow, 8)],
                    )

        def pair_body(i, carry):
            k = k0 + i * 2
            cp_b = stage(k + 1, idx_b, rows_b, sem_b)
            pltpu.make_async_copy(tab_hbm.at[pl.ds(0, BLK)], rows_a, sem_a).wait()
            consume(k, rows_a)

            @pl.when(i < PER_W // 2 - 1)
            def _():
                stage(k + 2, idx_a, rows_a, sem_a)

            cp_b.wait()
            consume(k + 1, rows_b)
            return carry

        stage(k0, idx_a, rows_a, sem_a)
        lax.fori_loop(0, PER_W // 2, pair_body, 0)

    return fused_kernel


_fused = _make_fused_kernel()


def kernel(mode, word_inputs, word_seq_lengths, table):
    idx_f = word_inputs.astype(jnp.int32).T.reshape(N)
    out_rows = _fused(table, idx_f)
    return out_rows.reshape(L, ET, B // 128, 8, 128).transpose(
        2, 4, 0, 1, 3
    ).reshape(B, L, EMB)


# speedup vs baseline: 2.8920x; 1.0806x over previous
import functools

import jax
import jax.numpy as jnp
from jax import lax
from jax.experimental import pallas as pl
from jax.experimental.pallas import tpu as pltpu
from jax.experimental.pallas import tpu_sc as plsc

VOCAB = 1000000
EMB = 64
B = 4096
L = 200
N = B * L

NC = 2
NS = 16
NW = NC * NS
BLK = 512
TASKS = N // BLK
PER_W = TASKS // NW
ET = EMB // 8
BT = BLK // 128
SROWS = ET * BT * 8
SPAD = 130
OROWS = L * EMB * B // 128


def _make_fused_kernel():
    mesh = plsc.VectorSubcoreMesh(core_axis_name="c", subcore_axis_name="s")

    @functools.partial(
        pl.kernel,
        mesh=mesh,
        out_type=jax.ShapeDtypeStruct((OROWS, 128), jnp.float32),
        compiler_params=pltpu.CompilerParams(
            use_tc_tiling_on_sc=False, needs_layout_passes=False
        ),
        scratch_types=[
            pltpu.VMEM((BLK,), jnp.int32),
            pltpu.VMEM((BLK,), jnp.int32),
            pltpu.VMEM((BLK, EMB), jnp.float32),
            pltpu.VMEM((BLK, EMB), jnp.float32),
            pltpu.VMEM((SROWS, SPAD), jnp.float32),
            pltpu.SemaphoreType.DMA,
            pltpu.SemaphoreType.DMA,
        ],
    )
    def fused_kernel(tab_hbm, idx_hbm, out_hbm, idx_a, idx_b, rows_a, rows_b,
                     stg, sem_a, sem_b):
        wid = lax.axis_index("s") * NC + lax.axis_index("c")
        k0 = wid * PER_W
        io16 = lax.iota(jnp.int32, 16)
        rowc = []
        for e0 in range(0, EMB, 16):
            e16 = e0 + io16
            rowc.append(lax.shift_right_logical(e16, 3) * (BT * 8) + (e16 & 7))

        def stage(k, idx_v, rows_v, sem):
            pltpu.sync_copy(idx_hbm.at[pl.ds(k * BLK, BLK)], idx_v)
            return pltpu.async_copy(tab_hbm.at[idx_v], rows_v, sem)

        def consume(k, rows_v):
            @plsc.parallel_loop(0, BLK, unroll=8)
            def _(j):
                bt8 = lax.shift_right_logical(j, 7) * 8
                col = jnp.full((16,), 0, jnp.int32) + (j & 127)
                for q in range(EMB // 16):
                    vec = rows_v[j, pl.ds(q * 16, 16)]
                    plsc.store_scatter(stg, [rowc[q] + bt8, col], vec)

            l = lax.shift_right_logical(k, 3)
            blk = k & 7
            for et in range(ET):
                drow = l * (EMB * B // 128) + et * (8 * B // 128) + blk * 32
                pltpu.sync_copy(
                    stg.at[pl.ds(et * 32, 32), pl.ds(0, 128)],
                    out_hbm.at[pl.ds(drow, 32)],
                )

        def pair_body(i, carry):
            k = k0 + i * 2
            cp_b = stage(k + 1, idx_b, rows_b, sem_b)
            pltpu.make_async_copy(tab_hbm.at[pl.ds(0, BLK)], rows_a, sem_a).wait()
            consume(k, rows_a)

            @pl.when(i < PER_W // 2 - 1)
            def _():
                stage(k + 2, idx_a, rows_a, sem_a)

            cp_b.wait()
            consume(k + 1, rows_b)
            return carry

        stage(k0, idx_a, rows_a, sem_a)
        lax.fori_loop(0, PER_W // 2, pair_body, 0)

    return fused_kernel


_fused = _make_fused_kernel()


def kernel(mode, word_inputs, word_seq_lengths, table):
    idx_f = word_inputs.astype(jnp.int32).T.reshape(N)
    out_rows = _fused(table, idx_f)
    return out_rows.reshape(L, ET, B // 128, 8, 128).transpose(
        2, 4, 0, 1, 3
    ).reshape(B, L, EMB)


# speedup vs baseline: 2.9001x; 1.0028x over previous
import functools

import jax
import jax.numpy as jnp
from jax import lax
from jax.experimental import pallas as pl
from jax.experimental.pallas import tpu as pltpu
from jax.experimental.pallas import tpu_sc as plsc

VOCAB = 1000000
EMB = 64
B = 4096
L = 200
N = B * L

NC = 2
NS = 16
NW = NC * NS
BLK = 512
TASKS = N // BLK
PER_W = TASKS // NW
ET = EMB // 8
BT = BLK // 128
SROWS = ET * BT * 8
SPAD = 130
OROWS = L * EMB * B // 128


def _make_fused_kernel():
    mesh = plsc.VectorSubcoreMesh(core_axis_name="c", subcore_axis_name="s")

    @functools.partial(
        pl.kernel,
        mesh=mesh,
        out_type=jax.ShapeDtypeStruct((OROWS, 128), jnp.float32),
        compiler_params=pltpu.CompilerParams(
            use_tc_tiling_on_sc=False, needs_layout_passes=False
        ),
        scratch_types=[
            pltpu.VMEM((BLK,), jnp.int32),
            pltpu.VMEM((BLK,), jnp.int32),
            pltpu.VMEM((BLK, EMB), jnp.float32),
            pltpu.VMEM((BLK, EMB), jnp.float32),
            pltpu.VMEM((SROWS, SPAD), jnp.float32),
            pltpu.SemaphoreType.DMA,
            pltpu.SemaphoreType.DMA,
        ],
    )
    def fused_kernel(tab_hbm, idx_hbm, out_hbm, idx_a, idx_b, rows_a, rows_b,
                     stg, sem_a, sem_b):
        wid = lax.axis_index("s") * NC + lax.axis_index("c")
        k0 = wid * PER_W
        io16 = lax.iota(jnp.int32, 16)
        rowc = []
        for e0 in range(0, EMB, 16):
            e16 = e0 + io16
            rowc.append(lax.shift_right_logical(e16, 3) * (BT * 8) + (e16 & 7))

        def stage(k, idx_v, rows_v, sem):
            pltpu.sync_copy(idx_hbm.at[pl.ds(k * BLK, BLK)], idx_v)
            return pltpu.async_copy(tab_hbm.at[idx_v], rows_v, sem)

        def consume(k, rows_v):
            @plsc.parallel_loop(0, BLK, unroll=16)
            def _(j):
                bt8 = lax.shift_right_logical(j, 7) * 8
                col = jnp.full((16,), 0, jnp.int32) + (j & 127)
                for q in range(EMB // 16):
                    vec = rows_v[j, pl.ds(q * 16, 16)]
                    plsc.store_scatter(stg, [rowc[q] + bt8, col], vec)

            l = lax.shift_right_logical(k, 3)
            blk = k & 7
            for et in range(ET):
                drow = l * (EMB * B // 128) + et * (8 * B // 128) + blk * 32
                pltpu.sync_copy(
                    stg.at[pl.ds(et * 32, 32), pl.ds(0, 128)],
                    out_hbm.at[pl.ds(drow, 32)],
                )

        def pair_body(i, carry):
            k = k0 + i * 2
            cp_b = stage(k + 1, idx_b, rows_b, sem_b)
            pltpu.make_async_copy(tab_hbm.at[pl.ds(0, BLK)], rows_a, sem_a).wait()
            consume(k, rows_a)

            @pl.when(i < PER_W // 2 - 1)
            def _():
                stage(k + 2, idx_a, rows_a, sem_a)

            cp_b.wait()
            consume(k + 1, rows_b)
            return carry

        stage(k0, idx_a, rows_a, sem_a)
        lax.fori_loop(0, PER_W // 2, pair_body, 0)

    return fused_kernel


_fused = _make_fused_kernel()


def kernel(mode, word_inputs, word_seq_lengths, table):
    idx_f = word_inputs.astype(jnp.int32).T.reshape(N)
    out_rows = _fused(table, idx_f)
    return out_rows.reshape(L, ET, B // 128, 8, 128).transpose(
        2, 4, 0, 1, 3
    ).reshape(B, L, EMB)
